# baseline (speedup 1.0000x reference)
import functools

import jax
import jax.numpy as jnp
from jax import lax
from jax.experimental import pallas as pl
from jax.experimental.pallas import tpu as pltpu
from jax.experimental.pallas import tpu_sc as plsc

N = 10000
E = 320000
NC = 2
NS = 16
CH = 128
EPAD = 327680
NACC = 10112

_MESH = plsc.VectorSubcoreMesh(
    core_axis_name="c", subcore_axis_name="s", num_cores=NC, num_subcores=NS
)


def _agg_body(edge_split, d, nch, NBUF, n_passes, IB, *refs):
    nt = NC * n_passes
    tables = refs[:nt]
    src, dst, zeros, out = refs[nt : nt + 4]
    tbl, acc, src_all, dst_all, rows = refs[nt + 4 : nt + 9]
    gsems = refs[nt + 9 : nt + 9 + NBUF]
    ssems = refs[nt + 9 + NBUF :]
    c = lax.axis_index("c")
    s = lax.axis_index("s")

    zrows = NACC // NS
    wrows = 624
    rem = N - NS * wrows
    ngrp = IB // NBUF

    def run(table_hbm, ch0):
        srcs = [tbl] * (NBUF - 1) + [table_hbm]

        def blk(bi, carry):
            b0 = ch0 + bi * IB
            pltpu.sync_copy(src.at[pl.ds(b0, IB)], src_all)
            pltpu.sync_copy(dst.at[pl.ds(b0, IB)], dst_all)

            for k in range(NBUF):
                pltpu.async_copy(srcs[k].at[src_all.at[k]], rows.at[k], gsems[k])

            def step(p, carry2):
                base = p * NBUF
                for k in range(NBUF):
                    pltpu.make_async_copy(
                        srcs[k].at[src_all.at[base + k]], rows.at[k], gsems[k]
                    ).wait()
                    pltpu.async_copy(
                        rows.at[k], acc.at[dst_all.at[base + k]], ssems[k], add=True
                    )
                for k in range(NBUF):
                    pltpu.make_async_copy(
                        rows.at[k], acc.at[dst_all.at[base + k]], ssems[k]
                    ).wait()

                    def _issue(k=k, nb=base + NBUF):
                        pltpu.async_copy(
                            srcs[k].at[src_all.at[nb + k]], rows.at[k], gsems[k]
                        )

                    pl.when(p + 1 < ngrp)(_issue)
                return carry2

            lax.fori_loop(0, ngrp, step, 0)
            return carry

        lax.fori_loop(0, nch // IB, blk, 0)

    for cc in range(NC):
        def _core(cc=cc):
            for t in range(n_passes):
                tb = tables[cc * n_passes + t]
                pltpu.sync_copy(
                    tb.at[pl.ds(s * wrows, wrows)], tbl.at[pl.ds(s * wrows, wrows)]
                )
                pltpu.sync_copy(
                    zeros.at[pl.ds(s * zrows, zrows)], acc.at[pl.ds(s * zrows, zrows)]
                )
                def _stail(tb=tb):
                    pltpu.sync_copy(
                        tb.at[pl.ds(NS * wrows, rem)], tbl.at[pl.ds(NS * wrows, rem)]
                    )
                pl.when(s == 0)(_stail)
                plsc.subcore_barrier()

                if edge_split:
                    run(tbl, (cc * NS + s) * nch)
                else:
                    run(tbl, s * nch)

                plsc.subcore_barrier()
                pltpu.sync_copy(
                    acc.at[pl.ds(s * wrows, wrows)],
                    out.at[cc * n_passes + t, pl.ds(s * wrows, wrows)],
                )
                def _wtail(ot=cc * n_passes + t):
                    pltpu.sync_copy(
                        acc.at[pl.ds(NS * wrows, rem)],
                        out.at[ot, pl.ds(NS * wrows, rem)],
                    )
                pl.when(s == 0)(_wtail)
                if t + 1 < n_passes:
                    plsc.subcore_barrier()
        pl.when(c == cc)(_core)


def _make_agg(edge_split, d, n_passes):
    per_core = EPAD // NC if edge_split else EPAD
    nch = per_core // NS // CH
    NBUF = 4
    IB = 32 if nch % 32 == 0 else 16
    body = functools.partial(_agg_body, edge_split, d, nch, NBUF, n_passes, IB)
    return pl.kernel(
        body,
        out_type=jax.ShapeDtypeStruct((NC * n_passes, N, d), jnp.float32),
        mesh=_MESH,
        scratch_types=[
            pltpu.VMEM_SHARED((N, d), jnp.float32),
            pltpu.VMEM_SHARED((NACC, d), jnp.float32),
            pltpu.VMEM((32 if nch % 32 == 0 else 16, CH), jnp.int32),
            pltpu.VMEM((32 if nch % 32 == 0 else 16, CH), jnp.int32),
            pltpu.VMEM((NBUF, CH, d), jnp.float32),
        ]
        + [pltpu.SemaphoreType.DMA] * (2 * NBUF),
        compiler_params=pltpu.CompilerParams(use_tc_tiling_on_sc=False),
    )


_agg1 = _make_agg(False, 64, 1)
_agg2 = _make_agg(False, 64, 2)
_agg3 = _make_agg(True, 48, 1)


_BM = 1000


def _mm1_body(p_ref, w_ref, b_ref, *o_refs):
    h = (
        jnp.dot(p_ref[0], w_ref[0], preferred_element_type=jnp.float32)
        + jnp.dot(p_ref[1], w_ref[1], preferred_element_type=jnp.float32)
        + b_ref[...]
    )
    h = jnp.maximum(h, 0.0)
    for q in range(4):
        o_refs[q][...] = h[:, 64 * q : 64 * (q + 1)]


def _mm1(p, W1r, b1r):
    return pl.pallas_call(
        _mm1_body,
        grid=(N // _BM,),
        in_specs=[
            pl.BlockSpec((NC, _BM, 64), lambda i: (0, i, 0)),
            pl.BlockSpec((2, 64, 256), lambda i: (0, 0, 0)),
            pl.BlockSpec((1, 256), lambda i: (0, 0)),
        ],
        out_specs=[pl.BlockSpec((_BM, 64), lambda i: (i, 0))] * 4,
        out_shape=[jax.ShapeDtypeStruct((N, 64), jnp.float32)] * 4,
    )(p, W1r, b1r)


def _mm23_body(a_ref, w2_ref, w3_ref, b2_ref, z_ref, z2_ref):
    h = (
        jnp.dot(a_ref[0], w2_ref[0], preferred_element_type=jnp.float32)
        + jnp.dot(a_ref[1], w2_ref[1], preferred_element_type=jnp.float32)
        + jnp.dot(a_ref[2], w2_ref[2], preferred_element_type=jnp.float32)
        + jnp.dot(a_ref[3], w2_ref[3], preferred_element_type=jnp.float32)
        + b2_ref[...]
    )
    h = jnp.maximum(h, 0.0)
    z = jnp.dot(h, w3_ref[...], preferred_element_type=jnp.float32)
    z_ref[...] = z
    z2_ref[...] = z


def _mm23(agg2, W2r, W3p, b2r):
    return pl.pallas_call(
        _mm23_body,
        grid=(N // _BM,),
        in_specs=[
            pl.BlockSpec((4, _BM, 64), lambda i: (0, i, 0)),
            pl.BlockSpec((4, 64, 256), lambda i: (0, 0, 0)),
            pl.BlockSpec((256, 48), lambda i: (0, 0)),
            pl.BlockSpec((1, 256), lambda i: (0, 0)),
        ],
        out_specs=[
            pl.BlockSpec((_BM, 48), lambda i: (i, 0)),
            pl.BlockSpec((_BM, 48), lambda i: (i, 0)),
        ],
        out_shape=[jax.ShapeDtypeStruct((N, 48), jnp.float32)] * 2,
    )(agg2, W2r, W3p, b2r)


def _fin_body(zz_ref, b3_ref, o_ref):
    t = zz_ref[0][:, :40] + zz_ref[1][:, :40] + b3_ref[...]
    m = jnp.max(t, axis=-1, keepdims=True)
    e = jnp.exp(t - m)
    lse = jnp.log(jnp.sum(e, axis=-1, keepdims=True))
    o_ref[...] = t - m - lse


def _fin(zz, b3r):
    return pl.pallas_call(
        _fin_body,
        grid=(N // _BM,),
        in_specs=[
            pl.BlockSpec((NC, _BM, 48), lambda i: (0, i, 0)),
            pl.BlockSpec((1, 40), lambda i: (0, 0)),
        ],
        out_specs=pl.BlockSpec((_BM, 40), lambda i: (i, 0)),
        out_shape=jax.ShapeDtypeStruct((N, 40), jnp.float32),
    )(zz, b3r)


def kernel(x, edge_index, W1, b1, W2, b2, W3, b3):
    src = edge_index[0]
    dst = edge_index[1]
    pad = EPAD - E
    srcp = jnp.concatenate([src, jnp.zeros((pad,), jnp.int32)]).reshape(EPAD // CH, CH)
    trash = N + jnp.arange(pad, dtype=jnp.int32) % (NACC - N)
    dstp = jnp.concatenate([dst, trash]).reshape(EPAD // CH, CH)
    z64 = jnp.zeros((NACC, 64), jnp.float32)
    z48 = jnp.zeros((NACC, 48), jnp.float32)
    W1r = W1.reshape(2, 64, 256)
    W2r = W2.reshape(4, 64, 256)
    W3p = jnp.pad(W3, ((0, 0), (0, 8)))
    xa = x[:, :64]
    xb = x[:, 64:]

    agg1 = _agg1(xa, xb, srcp, dstp, z64)
    h1q = _mm1(agg1, W1r, b1.reshape(1, 256))
    agg2 = _agg2(*h1q, srcp, dstp, z64)
    z, z2 = _mm23(agg2, W2r, W3p, b2.reshape(1, 256))
    agg3 = _agg3(z, z2, srcp, dstp, z48)
    return _fin(agg3, b3.reshape(1, 40))


# speedup vs baseline: 1.0166x; 1.0166x over previous
import functools

import jax
import jax.numpy as jnp
import numpy as np
from jax import lax
from jax.experimental import pallas as pl
from jax.experimental.pallas import tpu as pltpu
from jax.experimental.pallas import tpu_sc as plsc

N = 10000
E = 320000
NC = 2
NS = 16
CH = 128
EPAD = 327680
NACC = 10112

_MESH = plsc.VectorSubcoreMesh(
    core_axis_name="c", subcore_axis_name="s", num_cores=NC, num_subcores=NS
)


def _agg_body(edge_split, d, nch, NBUF, n_passes, IB, *refs):
    nt = NC * n_passes
    tables = refs[:nt]
    src, dst, zeros, out = refs[nt : nt + 4]
    tbl, acc, src_all, dst_all, rows = refs[nt + 4 : nt + 9]
    gsems = refs[nt + 9 : nt + 9 + NBUF]
    ssems = refs[nt + 9 + NBUF :]
    c = lax.axis_index("c")
    s = lax.axis_index("s")

    zrows = NACC // NS
    wrows = 624
    rem = N - NS * wrows
    ngrp = IB // NBUF

    def run(table_hbm, ch0):
        srcs = [tbl] * (NBUF - 1) + [table_hbm]

        def blk(bi, carry):
            b0 = ch0 + bi * IB
            pltpu.sync_copy(src.at[pl.ds(b0, IB)], src_all)
            pltpu.sync_copy(dst.at[pl.ds(b0, IB)], dst_all)

            for k in range(NBUF):
                pltpu.async_copy(srcs[k].at[src_all.at[k]], rows.at[k], gsems[k])

            def step(p, carry2):
                base = p * NBUF
                for k in range(NBUF):
                    pltpu.make_async_copy(
                        srcs[k].at[src_all.at[base + k]], rows.at[k], gsems[k]
                    ).wait()
                    pltpu.async_copy(
                        rows.at[k], acc.at[dst_all.at[base + k]], ssems[k], add=True
                    )
                for k in range(NBUF):
                    pltpu.make_async_copy(
                        rows.at[k], acc.at[dst_all.at[base + k]], ssems[k]
                    ).wait()

                    def _issue(k=k, nb=base + NBUF):
                        pltpu.async_copy(
                            srcs[k].at[src_all.at[nb + k]], rows.at[k], gsems[k]
                        )

                    pl.when(p + 1 < ngrp)(_issue)
                return carry2

            lax.fori_loop(0, ngrp, step, 0)
            return carry

        lax.fori_loop(0, nch // IB, blk, 0)

    for cc in range(NC):
        def _core(cc=cc):
            for t in range(n_passes):
                tb = tables[cc * n_passes + t]
                pltpu.sync_copy(
                    tb.at[pl.ds(s * wrows, wrows)], tbl.at[pl.ds(s * wrows, wrows)]
                )
                pltpu.sync_copy(
                    zeros.at[pl.ds(s * zrows, zrows)], acc.at[pl.ds(s * zrows, zrows)]
                )
                def _stail(tb=tb):
                    pltpu.sync_copy(
                        tb.at[pl.ds(NS * wrows, rem)], tbl.at[pl.ds(NS * wrows, rem)]
                    )
                pl.when(s == 0)(_stail)
                plsc.subcore_barrier()

                if edge_split:
                    run(tbl, (cc * NS + s) * nch)
                else:
                    run(tbl, s * nch)

                plsc.subcore_barrier()
                pltpu.sync_copy(
                    acc.at[pl.ds(s * wrows, wrows)],
                    out.at[cc * n_passes + t, pl.ds(s * wrows, wrows)],
                )
                def _wtail(ot=cc * n_passes + t):
                    pltpu.sync_copy(
                        acc.at[pl.ds(NS * wrows, rem)],
                        out.at[ot, pl.ds(NS * wrows, rem)],
                    )
                pl.when(s == 0)(_wtail)
                if t + 1 < n_passes:
                    plsc.subcore_barrier()
        pl.when(c == cc)(_core)


def _make_agg(edge_split, d, n_passes):
    per_core = EPAD // NC if edge_split else EPAD
    nch = per_core // NS // CH
    NBUF = 4
    IB = 16
    body = functools.partial(_agg_body, edge_split, d, nch, NBUF, n_passes, IB)
    return pl.kernel(
        body,
        out_type=jax.ShapeDtypeStruct((NC * n_passes, N, d), jnp.float32),
        mesh=_MESH,
        scratch_types=[
            pltpu.VMEM_SHARED((N, d), jnp.float32),
            pltpu.VMEM_SHARED((NACC, d), jnp.float32),
            pltpu.VMEM((16, CH), jnp.int32),
            pltpu.VMEM((16, CH), jnp.int32),
            pltpu.VMEM((NBUF, CH, d), jnp.float32),
        ]
        + [pltpu.SemaphoreType.DMA] * (2 * NBUF),
        compiler_params=pltpu.CompilerParams(use_tc_tiling_on_sc=False),
    )


_agg1 = _make_agg(False, 64, 1)
_agg2 = _make_agg(False, 64, 2)
_agg3 = _make_agg(True, 48, 1)


_BM = 1000

_SRC_PAD = np.zeros((EPAD - E,), np.int32)
_TRASH = (N + np.arange(EPAD - E, dtype=np.int32) % (NACC - N)).astype(np.int32)


def _mm1_body(p_ref, w_ref, b_ref, *o_refs):
    h = (
        jnp.dot(p_ref[0], w_ref[0], preferred_element_type=jnp.float32)
        + jnp.dot(p_ref[1], w_ref[1], preferred_element_type=jnp.float32)
        + b_ref[...]
    )
    h = jnp.maximum(h, 0.0)
    for q in range(4):
        o_refs[q][...] = h[:, 64 * q : 64 * (q + 1)]


def _mm1(p, W1r, b1r):
    return pl.pallas_call(
        _mm1_body,
        grid=(N // _BM,),
        in_specs=[
            pl.BlockSpec((NC, _BM, 64), lambda i: (0, i, 0)),
            pl.BlockSpec((2, 64, 256), lambda i: (0, 0, 0)),
            pl.BlockSpec((1, 256), lambda i: (0, 0)),
        ],
        out_specs=[pl.BlockSpec((_BM, 64), lambda i: (i, 0))] * 4,
        out_shape=[jax.ShapeDtypeStruct((N, 64), jnp.float32)] * 4,
    )(p, W1r, b1r)


def _mm23_body(a_ref, w2_ref, w3_ref, b2_ref, z_ref, z2_ref):
    h = (
        jnp.dot(a_ref[0], w2_ref[0], preferred_element_type=jnp.float32)
        + jnp.dot(a_ref[1], w2_ref[1], preferred_element_type=jnp.float32)
        + jnp.dot(a_ref[2], w2_ref[2], preferred_element_type=jnp.float32)
        + jnp.dot(a_ref[3], w2_ref[3], preferred_element_type=jnp.float32)
        + b2_ref[...]
    )
    h = jnp.maximum(h, 0.0)
    z = jnp.dot(h, w3_ref[...], preferred_element_type=jnp.float32)
    z_ref[...] = z
    z2_ref[...] = z


def _mm23(agg2, W2r, W3p, b2r):
    return pl.pallas_call(
        _mm23_body,
        grid=(N // _BM,),
        in_specs=[
            pl.BlockSpec((4, _BM, 64), lambda i: (0, i, 0)),
            pl.BlockSpec((4, 64, 256), lambda i: (0, 0, 0)),
            pl.BlockSpec((256, 48), lambda i: (0, 0)),
            pl.BlockSpec((1, 256), lambda i: (0, 0)),
        ],
        out_specs=[
            pl.BlockSpec((_BM, 48), lambda i: (i, 0)),
            pl.BlockSpec((_BM, 48), lambda i: (i, 0)),
        ],
        out_shape=[jax.ShapeDtypeStruct((N, 48), jnp.float32)] * 2,
    )(agg2, W2r, W3p, b2r)


def _fin_body(zz_ref, b3_ref, o_ref):
    t = zz_ref[0][:, :40] + zz_ref[1][:, :40] + b3_ref[...]
    m = jnp.max(t, axis=-1, keepdims=True)
    e = jnp.exp(t - m)
    lse = jnp.log(jnp.sum(e, axis=-1, keepdims=True))
    o_ref[...] = t - m - lse


def _fin(zz, b3r):
    return pl.pallas_call(
        _fin_body,
        grid=(N // _BM,),
        in_specs=[
            pl.BlockSpec((NC, _BM, 48), lambda i: (0, i, 0)),
            pl.BlockSpec((1, 40), lambda i: (0, 0)),
        ],
        out_specs=pl.BlockSpec((_BM, 40), lambda i: (i, 0)),
        out_shape=jax.ShapeDtypeStruct((N, 40), jnp.float32),
    )(zz, b3r)


def kernel(x, edge_index, W1, b1, W2, b2, W3, b3):
    src = edge_index[0]
    dst = edge_index[1]
    pad = EPAD - E
    srcp = jnp.concatenate([src, _SRC_PAD]).reshape(EPAD // CH, CH)
    dstp = jnp.concatenate([dst, _TRASH]).reshape(EPAD // CH, CH)
    z64 = jnp.zeros((NACC, 64), jnp.float32)
    z48 = jnp.zeros((NACC, 48), jnp.float32)
    W1r = W1.reshape(2, 64, 256)
    W2r = W2.reshape(4, 64, 256)
    W3p = jnp.pad(W3, ((0, 0), (0, 8)))
    xa = x[:, :64]
    xb = x[:, 64:]

    agg1 = _agg1(xa, xb, srcp, dstp, z64)
    h1q = _mm1(agg1, W1r, b1.reshape(1, 256))
    agg2 = _agg2(*h1q, srcp, dstp, z64)
    z, z2 = _mm23(agg2, W2r, W3p, b2.reshape(1, 256))
    agg3 = _agg3(z, z2, srcp, dstp, z48)
    return _fin(agg3, b3.reshape(1, 40))
